# baseline (speedup 1.0000x reference)
import functools

import jax
import jax.numpy as jnp
from jax import lax
from jax.experimental import pallas as pl
from jax.experimental.pallas import tpu as pltpu
from jax.experimental.pallas import tpu_sc as plsc

NUM_ROWS = 100000
NUM_CLASSES = 1000
BATCH = 16384

NC = 2
NS = 16
NW = NC * NS
B_PER_W = BATCH // NW
CHUNK = 64
NCHUNK = B_PER_W // CHUNK


def _gather_body(idx_hbm, table_hbm, out_hbm, idx_v, buf0, buf1, sem0, sem1):
    wid = lax.axis_index("s") * NC + lax.axis_index("c")
    base = wid * B_PER_W

    pltpu.sync_copy(idx_hbm.at[pl.ds(base, B_PER_W)], idx_v)

    bufs = (buf0, buf1)
    sems = (sem0, sem1)

    def start(j):
        return pltpu.async_copy(
            table_hbm.at[idx_v.at[pl.ds(j * CHUNK, CHUNK)]],
            bufs[j % 2],
            sems[j % 2],
        )

    inflight = start(0)
    for j in range(NCHUNK):
        nxt = start(j + 1) if j + 1 < NCHUNK else None
        inflight.wait()
        pltpu.sync_copy(bufs[j % 2], out_hbm.at[pl.ds(base + j * CHUNK, CHUNK)])
        inflight = nxt


@jax.jit
def _gather(idx, activations):
    mesh = plsc.VectorSubcoreMesh(core_axis_name="c", subcore_axis_name="s")
    return pl.kernel(
        _gather_body,
        out_type=jax.ShapeDtypeStruct((BATCH, NUM_CLASSES), jnp.float32),
        mesh=mesh,
        scratch_types=[
            pltpu.VMEM((B_PER_W,), jnp.int32),
            pltpu.VMEM((CHUNK, NUM_CLASSES), jnp.float32),
            pltpu.VMEM((CHUNK, NUM_CLASSES), jnp.float32),
            pltpu.SemaphoreType.DMA,
            pltpu.SemaphoreType.DMA,
        ],
        compiler_params=pltpu.CompilerParams(use_tc_tiling_on_sc=False),
    )(idx, activations)


def kernel(idx, x, activations):
    del x
    return _gather(idx.astype(jnp.int32), activations)


# speedup vs baseline: 4.8723x; 4.8723x over previous
import jax
import jax.numpy as jnp
from jax import lax
from jax.experimental import pallas as pl
from jax.experimental.pallas import tpu as pltpu
from jax.experimental.pallas import tpu_sc as plsc

NUM_ROWS = 100000
NUM_CLASSES = 1000
BATCH = 16384

NC = 2
NS = 16
NW = NC * NS
B_PER_W = BATCH // NW
CHUNK = 32
NCHUNK = B_PER_W // CHUNK
NPAIR = NCHUNK // 2


def _gather_body(idx_hbm, table_hbm, out_hbm, idx_v, buf, sem0, sem1):
    wid = lax.axis_index("s") * NC + lax.axis_index("c")
    base = wid * B_PER_W

    pltpu.sync_copy(idx_hbm.at[pl.ds(base, B_PER_W)], idx_v)

    sems = (sem0, sem1)

    def issue(j, half):
        sem = sems[half]
        for c16 in range(CHUNK // 16):
            vec = idx_v[pl.ds(j * CHUNK + c16 * 16, 16)]
            for l in range(16):
                r = vec[l]
                pltpu.async_copy(
                    table_hbm.at[pl.ds(r, 1)],
                    buf.at[pl.ds(half * CHUNK + c16 * 16 + l, 1)],
                    sem,
                )

    def drain_and_write(j, half):
        pltpu.make_async_copy(
            table_hbm.at[pl.ds(0, CHUNK)],
            buf.at[pl.ds(half * CHUNK, CHUNK)],
            sems[half],
        ).wait()
        pltpu.sync_copy(
            buf.at[pl.ds(half * CHUNK, CHUNK)],
            out_hbm.at[pl.ds(base + j * CHUNK, CHUNK)],
        )

    issue(0, 0)

    def pair_body(t, carry):
        issue(2 * t + 1, 1)
        drain_and_write(2 * t, 0)

        @pl.when(t + 1 < NPAIR)
        def _():
            issue(2 * t + 2, 0)

        drain_and_write(2 * t + 1, 1)
        return carry

    lax.fori_loop(0, NPAIR, pair_body, 0)


@jax.jit
def _gather(idx, activations):
    mesh = plsc.VectorSubcoreMesh(core_axis_name="c", subcore_axis_name="s")
    return pl.kernel(
        _gather_body,
        out_type=jax.ShapeDtypeStruct((BATCH, NUM_CLASSES), jnp.float32),
        mesh=mesh,
        scratch_types=[
            pltpu.VMEM((B_PER_W,), jnp.int32),
            pltpu.VMEM((2 * CHUNK, NUM_CLASSES), jnp.float32),
            pltpu.SemaphoreType.DMA,
            pltpu.SemaphoreType.DMA,
        ],
        compiler_params=pltpu.CompilerParams(use_tc_tiling_on_sc=True),
    )(idx, activations)


def kernel(idx, x, activations):
    del x
    return _gather(idx.astype(jnp.int32), activations)
